# Initial kernel scaffold; baseline (speedup 1.0000x reference)
import jax
import jax.numpy as jnp
from jax.experimental import pallas as pl


def kernel(x, weights, indices, w1, w2, w3, sw1, sw2, sw3):
    raise NotImplementedError("write your pallas kernel here")


# speedup vs baseline: 2.0679x; 2.0679x over previous
import functools

import jax
import jax.numpy as jnp
from jax import lax
from jax.experimental import pallas as pl
from jax.experimental.pallas import tpu as pltpu
from jax.experimental.pallas import tpu_sc as plsc

DIM = 2048
INTER = 1408
E = 64
TOPK = 6
T = 2048
SHARED_INTER = 2 * INTER
N = T * TOPK

B = 256
NPAD = N + E * B
NBLK = NPAD // B
IC = 128
NI = INTER // IC
SIC = 256
NSI = SHARED_INTER // SIC

NW = 32
GC = 32
CT = 8


def _sc_gather_body(tok_hbm, x_hbm, out_hbm, idx_v, rows_v, sem):
    wid = lax.axis_index("s") * 2 + lax.axis_index("c")
    rows_per_w = NPAD // NW
    base = wid * rows_per_w

    def step(i, carry):
        off = base + i * GC
        pltpu.sync_copy(tok_hbm.at[pl.ds(off, GC)], idx_v)
        pltpu.async_copy(x_hbm.at[idx_v], rows_v, sem).wait()
        pltpu.sync_copy(rows_v, out_hbm.at[pl.ds(off, GC)])
        return carry

    lax.fori_loop(0, rows_per_w // GC, step, 0)


def _sc_gather(tok, x):
    mesh = plsc.VectorSubcoreMesh(core_axis_name="c", subcore_axis_name="s")
    f = functools.partial(
        pl.kernel,
        mesh=mesh,
        out_type=jax.ShapeDtypeStruct((NPAD, DIM), jnp.float32),
        scratch_types=[
            pltpu.VMEM((GC,), jnp.int32),
            pltpu.VMEM((GC, DIM), jnp.float32),
            pltpu.SemaphoreType.DMA,
        ],
    )(_sc_gather_body)
    return f(tok, x)


def _sc_combine_body(pos_hbm, outs_hbm, z_hbm, y_hbm, idx_v, rows_v, acc_v, sem):
    wid = lax.axis_index("s") * 2 + lax.axis_index("c")
    tok_per_w = T // NW
    base = wid * tok_per_w

    def step(i, carry):
        t0 = base + i * CT
        pltpu.sync_copy(pos_hbm.at[pl.ds(t0 * TOPK, CT * TOPK)], idx_v)
        pltpu.async_copy(outs_hbm.at[idx_v], rows_v, sem).wait()
        pltpu.sync_copy(z_hbm.at[pl.ds(t0, CT)], acc_v)

        def dstep(d, c2):
            for tt in range(CT):
                s = acc_v[tt, pl.ds(d * 16, 16)]
                for k in range(TOPK):
                    s = s + rows_v[tt * TOPK + k, pl.ds(d * 16, 16)]
                acc_v[tt, pl.ds(d * 16, 16)] = s
            return c2

        lax.fori_loop(0, DIM // 16, dstep, 0)
        pltpu.sync_copy(acc_v, y_hbm.at[pl.ds(t0, CT)])
        return carry

    lax.fori_loop(0, tok_per_w // CT, step, 0)


def _sc_combine(pos_flat, outs, z):
    mesh = plsc.VectorSubcoreMesh(core_axis_name="c", subcore_axis_name="s")
    f = functools.partial(
        pl.kernel,
        mesh=mesh,
        out_type=jax.ShapeDtypeStruct((T, DIM), jnp.float32),
        scratch_types=[
            pltpu.VMEM((CT * TOPK,), jnp.int32),
            pltpu.VMEM((CT * TOPK, DIM), jnp.float32),
            pltpu.VMEM((CT, DIM), jnp.float32),
            pltpu.SemaphoreType.DMA,
        ],
    )(_sc_combine_body)
    return f(pos_flat, outs, z)


def _gmm_kernel(be_ref, valid_ref, xs_ref, gate_ref, w1_ref, w3_ref, w2_ref,
                out_ref, acc_ref):
    i = pl.program_id(0)
    j = pl.program_id(1)

    @pl.when(j == 0)
    def _():
        acc_ref[...] = jnp.zeros_like(acc_ref)

    @pl.when(valid_ref[i] > 0)
    def _():
        xb = xs_ref[...]
        w1c = w1_ref[0]
        w3c = w3_ref[0]
        w2c = w2_ref[0]
        h1 = lax.dot_general(xb, w1c, (((1,), (1,)), ((), ())),
                             preferred_element_type=jnp.float32)
        h3 = lax.dot_general(xb, w3c, (((1,), (1,)), ((), ())),
                             preferred_element_type=jnp.float32)
        g = (h1 * jax.nn.sigmoid(h1)) * h3
        acc_ref[...] += lax.dot_general(g, w2c, (((1,), (1,)), ((), ())),
                                        preferred_element_type=jnp.float32)

    @pl.when(j == NI - 1)
    def _():
        out_ref[...] = acc_ref[...] * gate_ref[0, 0, :][:, None]


def _gmm(be, valid, xs, gates, w1, w3, w2):
    grid_spec = pltpu.PrefetchScalarGridSpec(
        num_scalar_prefetch=2,
        grid=(NBLK, NI),
        in_specs=[
            pl.BlockSpec((B, DIM),
                         lambda i, j, be, val: (jnp.where(val[i] > 0, i, 0), 0)),
            pl.BlockSpec((1, 1, B), lambda i, j, be, val: (i, 0, 0)),
            pl.BlockSpec((1, IC, DIM),
                         lambda i, j, be, val: (be[i], jnp.where(val[i] > 0, j, 0), 0)),
            pl.BlockSpec((1, IC, DIM),
                         lambda i, j, be, val: (be[i], jnp.where(val[i] > 0, j, 0), 0)),
            pl.BlockSpec((1, DIM, IC),
                         lambda i, j, be, val: (be[i], 0, jnp.where(val[i] > 0, j, 0))),
        ],
        out_specs=pl.BlockSpec((B, DIM), lambda i, j, be, val: (i, 0)),
        scratch_shapes=[pltpu.VMEM((B, DIM), jnp.float32)],
    )
    return pl.pallas_call(
        _gmm_kernel,
        grid_spec=grid_spec,
        out_shape=jax.ShapeDtypeStruct((NPAD, DIM), jnp.float32),
        compiler_params=pltpu.CompilerParams(
            dimension_semantics=("arbitrary", "arbitrary")),
    )(be, valid, xs, gates, w1, w3, w2)


def _shared_kernel(x_ref, sw1_ref, sw3_ref, sw2_ref, z_ref, acc_ref):
    j = pl.program_id(1)

    @pl.when(j == 0)
    def _():
        acc_ref[...] = jnp.zeros_like(acc_ref)

    xb = x_ref[...]
    h1 = lax.dot_general(xb, sw1_ref[...], (((1,), (1,)), ((), ())),
                         preferred_element_type=jnp.float32)
    h3 = lax.dot_general(xb, sw3_ref[...], (((1,), (1,)), ((), ())),
                         preferred_element_type=jnp.float32)
    g = (h1 * jax.nn.sigmoid(h1)) * h3
    acc_ref[...] += lax.dot_general(g, sw2_ref[...], (((1,), (1,)), ((), ())),
                                    preferred_element_type=jnp.float32)

    @pl.when(j == NSI - 1)
    def _():
        z_ref[...] = acc_ref[...]


def _shared_mlp(x, sw1, sw3, sw2):
    bt = 256
    return pl.pallas_call(
        _shared_kernel,
        grid=(T // bt, NSI),
        in_specs=[
            pl.BlockSpec((bt, DIM), lambda i, j: (i, 0)),
            pl.BlockSpec((SIC, DIM), lambda i, j: (j, 0)),
            pl.BlockSpec((SIC, DIM), lambda i, j: (j, 0)),
            pl.BlockSpec((DIM, SIC), lambda i, j: (0, j)),
        ],
        out_specs=pl.BlockSpec((bt, DIM), lambda i, j: (i, 0)),
        out_shape=jax.ShapeDtypeStruct((T, DIM), jnp.float32),
        scratch_shapes=[pltpu.VMEM((bt, DIM), jnp.float32)],
        compiler_params=pltpu.CompilerParams(
            dimension_semantics=("arbitrary", "arbitrary")),
    )(x, sw1, sw3, sw2)


def _route(indices, weights):
    ids = indices.reshape(-1).astype(jnp.int32)
    gvals = weights.reshape(-1).astype(jnp.float32)
    order = jnp.argsort(ids)
    es = ids[order]
    counts = jnp.bincount(ids, length=E).astype(jnp.int32)
    pc = ((counts + B - 1) // B) * B
    offs = jnp.concatenate([jnp.zeros((1,), jnp.int32),
                            jnp.cumsum(pc)[:-1].astype(jnp.int32)])
    ccounts = jnp.concatenate([jnp.zeros((1,), jnp.int32),
                               jnp.cumsum(counts)[:-1].astype(jnp.int32)])
    ranks = jnp.arange(N, dtype=jnp.int32) - ccounts[es]
    dest = offs[es] + ranks
    tok_sorted = jnp.zeros((NPAD,), jnp.int32).at[dest].set(
        (order // TOPK).astype(jnp.int32))
    gate_sorted = jnp.zeros((NPAD,), jnp.float32).at[dest].set(gvals[order])
    pos_flat = jnp.zeros((N,), jnp.int32).at[order].set(dest)
    total = jnp.sum(pc).astype(jnp.int32)
    bstart = jnp.arange(NBLK, dtype=jnp.int32) * B
    valid = (bstart < total).astype(jnp.int32)
    bclamp = jnp.minimum(bstart, total - 1)
    be = (jnp.searchsorted(offs, bclamp, side="right") - 1).astype(jnp.int32)
    return tok_sorted, gate_sorted, pos_flat, be, valid


def kernel(x, weights, indices, w1, w2, w3, sw1, sw2, sw3):
    tok_sorted, gate_sorted, pos_flat, be, valid = _route(indices, weights)
    xs = _sc_gather(tok_sorted, x)
    gates = gate_sorted.reshape(NBLK, 1, B)
    outs = _gmm(be, valid, xs, gates, w1, w3, w2)
    z = _shared_mlp(x, sw1, sw3, sw2)
    y = _sc_combine(pos_flat, outs, z)
    return y
